# baseline (speedup 1.0000x reference)
import jax
import jax.numpy as jnp
from jax.experimental import pallas as pl

_V = 32000
_MASK_ID = 31999
_P_UNIFORM = 0.1
_U = _P_UNIFORM / _V

_BS = 64


def _gidd_block(z_ref, coef_ref, x_ref, o_ref):
    x = x_ref[...]
    bs, v = x.shape
    m = jnp.max(x, axis=1, keepdims=True)

    v_idx = jax.lax.broadcasted_iota(jnp.int32, (bs, v), 1)
    z = z_ref[...]
    onehot = v_idx == z
    x_z = jnp.sum(jnp.where(onehot, x, 0.0), axis=1, keepdims=True)

    e = jnp.exp(x - m)
    zsum_raw = jnp.sum(e, axis=1, keepdims=True)
    x_mask = x_ref[:, v - 1:v]
    e_mask = jnp.exp(x_mask - m)
    zsum = zsum_raw - e_mask

    coef = coef_ref[...]
    t = coef[:, 0:1]
    a_t = coef[:, 1:2]
    s = coef[:, 2:3]
    a_s = coef[:, 3:4]
    a_ts = coef[:, 4:5]
    c_ts = coef[:, 5:6]

    mask_hit = (z == _MASK_ID).astype(x.dtype)
    pi_z = _U + 0.9 * mask_hit
    p_z = jnp.where(mask_hit > 0.0, 0.0, jnp.exp(x_z - m)) / zsum
    q_zt = a_t * p_z + t * pi_z
    g0 = (pi_z * c_ts) / q_zt
    g1 = a_ts / q_zt

    c1 = a_s / zsum
    c2 = s * _U
    o_ref[...] = jnp.where(onehot, g0 + g1, g0) * (c1 * e + c2)
    o_ref[:, v - 1:v] = (g0 + g1 * mask_hit) * (c2 + s * 0.9)


def kernel(logits, z_t, t, s):
    B, S, V = logits.shape
    R = B * S
    x = logits.reshape(R, V)
    z = z_t.reshape(R, 1).astype(jnp.int32)

    a_t = 1.0 - t
    a_s = 1.0 - s
    a_ts = a_t / a_s
    c_ts = t - a_ts * s
    zero = jnp.zeros_like(t)
    coef_b = jnp.stack([t, a_t, s, a_s, a_ts, c_ts, zero, zero], axis=1)
    coef = jnp.broadcast_to(coef_b[:, None, :], (B, S, 8)).reshape(R, 8)

    out = pl.pallas_call(
        _gidd_block,
        grid=(R // _BS,),
        in_specs=[
            pl.BlockSpec((_BS, 1), lambda i: (i, 0)),
            pl.BlockSpec((_BS, 8), lambda i: (i, 0)),
            pl.BlockSpec((_BS, V), lambda i: (i, 0)),
        ],
        out_specs=pl.BlockSpec((_BS, V), lambda i: (i, 0)),
        out_shape=jax.ShapeDtypeStruct((R, V), jnp.float32),
    )(z, coef, x)
    return out.reshape(B, S, V)
